# baseline (speedup 1.0000x reference)
import functools

import jax
import jax.numpy as jnp
from jax import lax
from jax.experimental import pallas as pl
from jax.experimental.pallas import tpu as pltpu
from jax.experimental.pallas import tpu_sc as plsc

D = 64
B = 4096 * 200
NC, NS = 2, 16
NW = NC * NS
BPW = B // NW
CHUNK = 128
NCH = BPW // CHUNK

_mesh = plsc.VectorSubcoreMesh(core_axis_name="c", subcore_axis_name="s")


@functools.partial(
    pl.kernel,
    mesh=_mesh,
    out_type=jax.ShapeDtypeStruct((B, D), jnp.float32),
    compiler_params=pltpu.CompilerParams(use_tc_tiling_on_sc=False),
    scratch_types=[
        pltpu.VMEM((NCH, CHUNK), jnp.int32),
        pltpu.VMEM((2, CHUNK, D), jnp.float32),
        pltpu.SemaphoreType.DMA,
        pltpu.SemaphoreType.DMA,
    ],
)
def _emb_lookup(ids_hbm, table_hbm, out_hbm, idx_v, rows_v, gsem, ssem):
    wid = lax.axis_index("s") * NC + lax.axis_index("c")
    base = wid * BPW
    pltpu.sync_copy(ids_hbm.at[pl.ds(wid * NCH, NCH)], idx_v)

    def body(j, carry):
        slot = lax.rem(j, 2)
        gather = pltpu.async_copy(
            table_hbm.at[idx_v.at[j]], rows_v.at[slot], gsem)
        gather.wait()
        store = pltpu.async_copy(
            rows_v.at[slot], out_hbm.at[pl.ds(base + j * CHUNK, CHUNK)], ssem)
        store.wait()
        return carry

    lax.fori_loop(0, NCH, body, 0)


def kernel(token_ids, embed_mat):
    ids = token_ids.reshape(NW * NCH, CHUNK).astype(jnp.int32)
    out = _emb_lookup(ids, embed_mat)
    return out.reshape(token_ids.shape[0], token_ids.shape[1], D)


# speedup vs baseline: 1.1170x; 1.1170x over previous
import functools

import jax
import jax.numpy as jnp
from jax import lax
from jax.experimental import pallas as pl
from jax.experimental.pallas import tpu as pltpu
from jax.experimental.pallas import tpu_sc as plsc

D = 64
B = 4096 * 200
NC, NS = 2, 16
NW = NC * NS
BPW = B // NW
CHUNK = 128
NCH = BPW // CHUNK
NBUF = 8
NROUNDS = NCH // NBUF

_mesh = plsc.VectorSubcoreMesh(core_axis_name="c", subcore_axis_name="s")


@functools.partial(
    pl.kernel,
    mesh=_mesh,
    out_type=jax.ShapeDtypeStruct((B, D), jnp.float32),
    compiler_params=pltpu.CompilerParams(use_tc_tiling_on_sc=False),
    scratch_types=[
        pltpu.VMEM((NCH, CHUNK), jnp.int32),
        pltpu.VMEM((NBUF, CHUNK, D), jnp.float32),
        pltpu.SemaphoreType.DMA((NBUF,)),
        pltpu.SemaphoreType.DMA((NBUF,)),
    ],
)
def _emb_lookup(ids_hbm, table_hbm, out_hbm, idx_v, rows_v, gsem, ssem):
    wid = lax.axis_index("s") * NC + lax.axis_index("c")
    base = wid * BPW
    pltpu.sync_copy(ids_hbm.at[pl.ds(wid * NCH, NCH)], idx_v)

    def gather(c, b):
        pltpu.async_copy(table_hbm.at[idx_v.at[c]], rows_v.at[b], gsem.at[b])

    def gather_wait(b):
        pltpu.make_async_copy(table_hbm.at[idx_v.at[0]], rows_v.at[b],
                              gsem.at[b]).wait()

    def store(c, b):
        pltpu.async_copy(rows_v.at[b],
                         out_hbm.at[pl.ds(base + c * CHUNK, CHUNK)],
                         ssem.at[b])

    def store_wait(b):
        pltpu.make_async_copy(rows_v.at[b], out_hbm.at[pl.ds(base, CHUNK)],
                              ssem.at[b]).wait()

    for b in range(NBUF):
        gather(b, b)

    def body(r, carry):
        c0 = r * NBUF
        for b in range(NBUF):
            gather_wait(b)
            store(c0 + b, b)
        for b in range(NBUF):
            store_wait(b)
            gather(c0 + NBUF + b, b)
        return carry

    lax.fori_loop(0, NROUNDS - 1, body, 0)

    c0 = (NROUNDS - 1) * NBUF
    for b in range(NBUF):
        gather_wait(b)
        store(c0 + b, b)
    for b in range(NBUF):
        store_wait(b)


def kernel(token_ids, embed_mat):
    ids = token_ids.reshape(NW * NCH, CHUNK).astype(jnp.int32)
    out = _emb_lookup(ids, embed_mat)
    return out.reshape(token_ids.shape[0], token_ids.shape[1], D)
